# Initial kernel scaffold; baseline (speedup 1.0000x reference)
import jax
import jax.numpy as jnp
from jax.experimental import pallas as pl


def kernel(x):
    raise NotImplementedError("write your pallas kernel here")


# speedup vs baseline: 1.7683x; 1.7683x over previous
import jax
import jax.numpy as jnp
from jax.experimental import pallas as pl
from jax.experimental.pallas import tpu as pltpu


def _widen_lanes(y):
    h, w = y.shape
    chunks = []
    for j in range(0, w, 128):
        src = y[:, j:j + 128]
        n = min(128, w - j)
        idx = jax.lax.broadcasted_iota(jnp.int32, (h, 2 * n), 1) // 2
        chunks.append(jnp.take_along_axis(src, idx, axis=1))
    return jnp.concatenate(chunks, axis=1)


def _body(x_ref, o_ref):
    y = x_ref[0]
    h, w = y.shape
    wide = _widen_lanes(y)
    col = jax.lax.broadcasted_iota(jnp.int32, (2 * h, h), 1)
    row = jax.lax.broadcasted_iota(jnp.int32, (2 * h, h), 0)
    dup = (row // 2 == col).astype(y.dtype)
    full = jax.lax.dot_general(
        dup, wide, (((1,), (0,)), ((), ())),
        preferred_element_type=jnp.float32)
    o_ref[0] = full


def kernel(x):
    n, c, h, w = x.shape
    b = n * c
    xf = x.reshape(b, h, w)
    out = pl.pallas_call(
        _body,
        grid=(b,),
        in_specs=[pl.BlockSpec((1, h, w), lambda i: (i, 0, 0))],
        out_specs=pl.BlockSpec((1, 2 * h, 2 * w), lambda i: (i, 0, 0)),
        out_shape=jax.ShapeDtypeStruct((b, 2 * h, 2 * w), x.dtype),
    )(xf)
    return out.reshape(n, c, 2 * h, 2 * w)
